# baseline (speedup 1.0000x reference)
import functools

import jax
import jax.numpy as jnp
from jax.experimental import pallas as pl
from jax.experimental.pallas import tpu as pltpu

_DIM = 2048
_NE = 8
_TOPK = 2
_BLK = 512


def _router_body(x_ref, w_ref, idx_ref, wgt_ref):
    st = jax.lax.dot_general(
        w_ref[...], x_ref[...],
        (((1,), (1,)), ((), ())),
        preferred_element_type=jnp.float32,
    )
    m = jnp.max(st, axis=0, keepdims=True)
    e = jnp.exp(st - m)
    p = e * (1.0 / jnp.sum(e, axis=0, keepdims=True))

    iota = jax.lax.broadcasted_iota(jnp.int32, st.shape, 0)
    ninf = jnp.float32(-jnp.inf)
    big = jnp.int32(_NE)

    p1 = jnp.max(p, axis=0, keepdims=True)
    i1 = jnp.min(jnp.where(p == p1, iota, big), axis=0, keepdims=True)
    pm = jnp.where(iota == i1, ninf, p)
    p2 = jnp.max(pm, axis=0, keepdims=True)
    i2 = jnp.min(jnp.where(pm == p2, iota, big), axis=0, keepdims=True)

    w1 = jnp.max(jnp.where(iota == i1, st, ninf), axis=0, keepdims=True)
    w2 = jnp.max(jnp.where(iota == i2, st, ninf), axis=0, keepdims=True)

    idx_ref[...] = jnp.concatenate([i1, i2], axis=0)
    wgt_ref[...] = jnp.concatenate([w1, w2], axis=0)


@jax.jit
def kernel(x, W):
    T = x.shape[0]
    grid = (T // _BLK,)
    idx_t, wgt_t = pl.pallas_call(
        _router_body,
        grid=grid,
        in_specs=[
            pl.BlockSpec((_BLK, _DIM), lambda i: (i, 0)),
            pl.BlockSpec((_NE, _DIM), lambda i: (0, 0)),
        ],
        out_specs=[
            pl.BlockSpec((_TOPK, _BLK), lambda i: (0, i)),
            pl.BlockSpec((_TOPK, _BLK), lambda i: (0, i)),
        ],
        out_shape=[
            jax.ShapeDtypeStruct((_TOPK, T), jnp.int32),
            jax.ShapeDtypeStruct((_TOPK, T), jnp.float32),
        ],
    )(x, W)
    return idx_t.T, wgt_t.T
